# baseline (speedup 1.0000x reference)
import functools

import jax
import jax.numpy as jnp
from jax import lax
from jax.experimental import pallas as pl
from jax.experimental.pallas import tpu as pltpu
from jax.experimental.pallas import tpu_sc as plsc

B = 16
TOTAL = 16384
D = 256
SEG = TOTAL // B
N_STAGES = 2
STAGE_B = B // N_STAGES
NC = 2
NS = 16
NW = NC * NS
WPS = NW // STAGE_B
ROWS_W = SEG // WPS
CH = 128
NCH = ROWS_W // CH
LANES = 16
G = D // LANES

_sc_mesh = plsc.VectorSubcoreMesh(core_axis_name="c", subcore_axis_name="s")


def _make_sc_stats(seg_off):
    row_off = seg_off * SEG

    @functools.partial(
        pl.kernel,
        out_type=jax.ShapeDtypeStruct((STAGE_B, WPS, 3 * D), jnp.float32),
        mesh=_sc_mesh,
        scratch_types=[
            pltpu.VMEM((CH, D), jnp.float32),
            pltpu.VMEM((CH, D), jnp.float32),
            pltpu.VMEM((3 * D,), jnp.float32),
            pltpu.SemaphoreType.DMA,
            pltpu.SemaphoreType.DMA,
        ],
    )
    def _sc_stats(x_hbm, part_hbm, xv0, xv1, pv, sem0, sem1):
        c = lax.axis_index("c")
        sub = lax.axis_index("s")
        w = c * NS + sub
        row0 = row_off + w * ROWS_W

        bufs = (xv0, xv1)
        sems = (sem0, sem1)

        zero = jnp.zeros((LANES,), jnp.float32)
        pinf = jnp.full((LANES,), jnp.inf, jnp.float32)
        ninf = jnp.full((LANES,), -jnp.inf, jnp.float32)
        carry = (
            tuple(zero for _ in range(G)),
            tuple(pinf for _ in range(G)),
            tuple(ninf for _ in range(G)),
        )

        handles = [
            pltpu.async_copy(
                x_hbm.at[pl.ds(row0 + k * CH, CH)], bufs[k % 2], sems[k % 2]
            )
            for k in range(min(2, NCH))
        ]
        for k in range(NCH):
            cur = k % 2
            handles[cur].wait()
            xv = bufs[cur]

            def row_body(r, acc, xv=xv):
                sums, mns, mxs = acc
                new_s, new_n, new_x = [], [], []
                for g in range(G):
                    v = xv[r, pl.ds(g * LANES, LANES)]
                    new_s.append(sums[g] + v)
                    new_n.append(jnp.minimum(mns[g], v))
                    new_x.append(jnp.maximum(mxs[g], v))
                return (tuple(new_s), tuple(new_n), tuple(new_x))

            carry = lax.fori_loop(0, CH, row_body, carry)
            if k + 2 < NCH:
                handles[cur] = pltpu.async_copy(
                    x_hbm.at[pl.ds(row0 + (k + 2) * CH, CH)], bufs[cur], sems[cur]
                )

        sums, mns, mxs = carry
        for g in range(G):
            pv[pl.ds(g * LANES, LANES)] = sums[g]
            pv[pl.ds(D + g * LANES, LANES)] = mns[g]
            pv[pl.ds(2 * D + g * LANES, LANES)] = mxs[g]
        pltpu.sync_copy(pv, part_hbm.at[w // WPS, w % WPS])

    return _sc_stats


_sc_stats_calls = [_make_sc_stats(s * STAGE_B) for s in range(N_STAGES)]


def _asm_kernel(inv_ref, part_ref, x_ref, out_ref):
    i = pl.program_id(0)
    p = part_ref[0]
    inv = inv_ref[i]
    mean = jnp.sum(p[:, 0:D], axis=0, keepdims=True) * inv
    mn = jnp.min(p[:, D:2 * D], axis=0, keepdims=True)
    mx = jnp.max(p[:, 2 * D:3 * D], axis=0, keepdims=True)
    out_ref[:, 0:D] = jnp.broadcast_to(mean, (SEG, D))
    out_ref[:, D:2 * D] = jnp.broadcast_to(mn, (SEG, D))
    out_ref[:, 2 * D:3 * D] = jnp.broadcast_to(mx, (SEG, D))
    out_ref[:, 3 * D:4 * D] = x_ref[...]


def _asm_call(seg_off, inv_half, part, x_data, buf=None):
    in_specs = [
        pl.BlockSpec((1, WPS, 3 * D), lambda i, *_: (i, 0, 0)),
        pl.BlockSpec((SEG, D), lambda i, *_: (i + seg_off, 0)),
    ]
    operands = [inv_half, part, x_data]
    aliases = {}
    body = _asm_kernel
    if buf is not None:
        in_specs.append(pl.BlockSpec(memory_space=pltpu.MemorySpace.HBM))
        operands.append(buf)
        aliases = {3: 0}

        def body(inv_ref, part_ref, x_ref, buf_ref, out_ref):
            del buf_ref
            _asm_kernel(inv_ref, part_ref, x_ref, out_ref)

    return pl.pallas_call(
        body,
        grid_spec=pltpu.PrefetchScalarGridSpec(
            num_scalar_prefetch=1,
            grid=(STAGE_B,),
            in_specs=in_specs,
            out_specs=pl.BlockSpec((SEG, 4 * D), lambda i, *_: (i + seg_off, 0)),
        ),
        out_shape=jax.ShapeDtypeStruct((TOTAL, 4 * D), jnp.float32),
        input_output_aliases=aliases,
    )(*operands)


def kernel(x_data, row_splits):
    counts = (row_splits[1:] - row_splits[:-1]).astype(jnp.float32)
    inv_counts = 1.0 / counts
    parts = [sc(x_data) for sc in _sc_stats_calls]
    buf = None
    for s in range(N_STAGES):
        off = s * STAGE_B
        buf = _asm_call(off, inv_counts[off:off + STAGE_B], parts[s], x_data,
                        buf=buf)
    return buf


# speedup vs baseline: 1.0316x; 1.0316x over previous
import functools

import jax
import jax.numpy as jnp
from jax import lax
from jax.experimental import pallas as pl
from jax.experimental.pallas import tpu as pltpu
from jax.experimental.pallas import tpu_sc as plsc

B = 16
TOTAL = 16384
D = 256
SEG = TOTAL // B
N_STAGES = 2
STAGE_B = B // N_STAGES
NC = 2
NS = 16
NW = NC * NS
WPS = NW // STAGE_B
ROWS_W = SEG // WPS
CH = 128
NCH = ROWS_W // CH
LANES = 16
G = D // LANES

_sc_mesh = plsc.VectorSubcoreMesh(core_axis_name="c", subcore_axis_name="s")


def _make_sc_stats(seg_off):
    row_off = seg_off * SEG

    @functools.partial(
        pl.kernel,
        out_type=jax.ShapeDtypeStruct((STAGE_B, WPS, 3 * D), jnp.float32),
        mesh=_sc_mesh,
        scratch_types=[
            pltpu.VMEM((CH, D), jnp.float32),
            pltpu.VMEM((CH, D), jnp.float32),
            pltpu.VMEM((3 * D,), jnp.float32),
            pltpu.SemaphoreType.DMA,
            pltpu.SemaphoreType.DMA,
        ],
    )
    def _sc_stats(x_hbm, part_hbm, xv0, xv1, pv, sem0, sem1):
        c = lax.axis_index("c")
        sub = lax.axis_index("s")
        w = c * NS + sub
        row0 = row_off + w * ROWS_W

        bufs = (xv0, xv1)
        sems = (sem0, sem1)

        zero = jnp.zeros((LANES,), jnp.float32)
        pinf = jnp.full((LANES,), jnp.inf, jnp.float32)
        ninf = jnp.full((LANES,), -jnp.inf, jnp.float32)
        carry = (
            tuple(zero for _ in range(G)),
            tuple(pinf for _ in range(G)),
            tuple(ninf for _ in range(G)),
        )

        handles = [
            pltpu.async_copy(
                x_hbm.at[pl.ds(row0 + k * CH, CH)], bufs[k % 2], sems[k % 2]
            )
            for k in range(min(2, NCH))
        ]
        for k in range(NCH):
            cur = k % 2
            handles[cur].wait()
            xv = bufs[cur]

            def row_body(r, acc, xv=xv):
                sums, mns, mxs = acc
                new_s, new_n, new_x = [], [], []
                for g in range(G):
                    v = xv[r, pl.ds(g * LANES, LANES)]
                    new_s.append(sums[g] + v)
                    new_n.append(jnp.minimum(mns[g], v))
                    new_x.append(jnp.maximum(mxs[g], v))
                return (tuple(new_s), tuple(new_n), tuple(new_x))

            carry = lax.fori_loop(0, CH, row_body, carry)
            if k + 2 < NCH:
                handles[cur] = pltpu.async_copy(
                    x_hbm.at[pl.ds(row0 + (k + 2) * CH, CH)], bufs[cur], sems[cur]
                )

        sums, mns, mxs = carry
        for g in range(G):
            pv[pl.ds(g * LANES, LANES)] = sums[g]
            pv[pl.ds(D + g * LANES, LANES)] = mns[g]
            pv[pl.ds(2 * D + g * LANES, LANES)] = mxs[g]
        pltpu.sync_copy(pv, part_hbm.at[w // WPS, w % WPS])

    return _sc_stats


_sc_stats_calls = [_make_sc_stats(s * STAGE_B) for s in range(N_STAGES)]


def _asm_kernel(inv_ref, part_ref, x_ref, out_ref):
    i = pl.program_id(0)
    p = part_ref[0]
    inv = inv_ref[i]
    mean = jnp.sum(p[:, 0:D], axis=0, keepdims=True) * inv
    mn = jnp.min(p[:, D:2 * D], axis=0, keepdims=True)
    mx = jnp.max(p[:, 2 * D:3 * D], axis=0, keepdims=True)
    out_ref[:, 0:D] = jnp.broadcast_to(mean, (SEG, D))
    out_ref[:, D:2 * D] = jnp.broadcast_to(mn, (SEG, D))
    out_ref[:, 2 * D:3 * D] = jnp.broadcast_to(mx, (SEG, D))
    out_ref[:, 3 * D:4 * D] = x_ref[...]


def _xcopy_kernel(x_ref, out_ref):
    out_ref[...] = x_ref[...]


def _xcopy_call(x_data):
    return pl.pallas_call(
        _xcopy_kernel,
        grid=(STAGE_B,),
        in_specs=[pl.BlockSpec((SEG, D), lambda i: (i, 0))],
        out_specs=pl.BlockSpec((SEG, D), lambda i: (i, 3)),
        out_shape=jax.ShapeDtypeStruct((TOTAL, 4 * D), jnp.float32),
    )(x_data)


def _stats_only_kernel(inv_ref, part_ref, buf_ref, out_ref):
    del buf_ref
    i = pl.program_id(0)
    p = part_ref[0]
    inv = inv_ref[i]
    mean = jnp.sum(p[:, 0:D], axis=0, keepdims=True) * inv
    mn = jnp.min(p[:, D:2 * D], axis=0, keepdims=True)
    mx = jnp.max(p[:, 2 * D:3 * D], axis=0, keepdims=True)
    out_ref[:, 0:D] = jnp.broadcast_to(mean, (SEG, D))
    out_ref[:, D:2 * D] = jnp.broadcast_to(mn, (SEG, D))
    out_ref[:, 2 * D:3 * D] = jnp.broadcast_to(mx, (SEG, D))


def _stats_only_call(inv_half, part, buf):
    return pl.pallas_call(
        _stats_only_kernel,
        grid_spec=pltpu.PrefetchScalarGridSpec(
            num_scalar_prefetch=1,
            grid=(STAGE_B,),
            in_specs=[
                pl.BlockSpec((1, WPS, 3 * D), lambda i, *_: (i, 0, 0)),
                pl.BlockSpec(memory_space=pltpu.MemorySpace.HBM),
            ],
            out_specs=pl.BlockSpec((SEG, 3 * D), lambda i, *_: (i, 0)),
        ),
        out_shape=jax.ShapeDtypeStruct((TOTAL, 4 * D), jnp.float32),
        input_output_aliases={2: 0},
    )(inv_half, part, buf)


def _asm_call(seg_off, inv_half, part, x_data, buf=None):
    in_specs = [
        pl.BlockSpec((1, WPS, 3 * D), lambda i, *_: (i, 0, 0)),
        pl.BlockSpec((SEG, D), lambda i, *_: (i + seg_off, 0)),
    ]
    operands = [inv_half, part, x_data]
    aliases = {}
    body = _asm_kernel
    if buf is not None:
        in_specs.append(pl.BlockSpec(memory_space=pltpu.MemorySpace.HBM))
        operands.append(buf)
        aliases = {3: 0}

        def body(inv_ref, part_ref, x_ref, buf_ref, out_ref):
            del buf_ref
            _asm_kernel(inv_ref, part_ref, x_ref, out_ref)

    return pl.pallas_call(
        body,
        grid_spec=pltpu.PrefetchScalarGridSpec(
            num_scalar_prefetch=1,
            grid=(STAGE_B,),
            in_specs=in_specs,
            out_specs=pl.BlockSpec((SEG, 4 * D), lambda i, *_: (i + seg_off, 0)),
        ),
        out_shape=jax.ShapeDtypeStruct((TOTAL, 4 * D), jnp.float32),
        input_output_aliases=aliases,
    )(*operands)


def kernel(x_data, row_splits):
    counts = (row_splits[1:] - row_splits[:-1]).astype(jnp.float32)
    inv_counts = 1.0 / counts
    parts = [sc(x_data) for sc in _sc_stats_calls]
    buf = _xcopy_call(x_data)
    buf = _stats_only_call(inv_counts[0:STAGE_B], parts[0], buf)
    return _asm_call(STAGE_B, inv_counts[STAGE_B:B], parts[1], x_data, buf=buf)


# speedup vs baseline: 1.0469x; 1.0148x over previous
import functools

import jax
import jax.numpy as jnp
from jax import lax
from jax.experimental import pallas as pl
from jax.experimental.pallas import tpu as pltpu
from jax.experimental.pallas import tpu_sc as plsc

B = 16
TOTAL = 16384
D = 256
SEG = TOTAL // B
N_STAGES = 2
STAGE_B = B // N_STAGES
NC = 2
NS = 16
NW = NC * NS
WPS = NW // STAGE_B
ROWS_W = SEG // WPS
CH = 128
NCH = ROWS_W // CH
LANES = 16
G = D // LANES

_sc_mesh = plsc.VectorSubcoreMesh(core_axis_name="c", subcore_axis_name="s")


def _make_sc_stats(seg_off):
    row_off = seg_off * SEG

    @functools.partial(
        pl.kernel,
        out_type=jax.ShapeDtypeStruct((STAGE_B, WPS, 3 * D), jnp.float32),
        mesh=_sc_mesh,
        scratch_types=[
            pltpu.VMEM((CH, D), jnp.float32),
            pltpu.VMEM((CH, D), jnp.float32),
            pltpu.VMEM((3 * D,), jnp.float32),
            pltpu.SemaphoreType.DMA,
            pltpu.SemaphoreType.DMA,
        ],
    )
    def _sc_stats(x_hbm, part_hbm, xv0, xv1, pv, sem0, sem1):
        c = lax.axis_index("c")
        sub = lax.axis_index("s")
        w = c * NS + sub
        row0 = row_off + w * ROWS_W

        bufs = (xv0, xv1)
        sems = (sem0, sem1)

        zero = jnp.zeros((LANES,), jnp.float32)
        pinf = jnp.full((LANES,), jnp.inf, jnp.float32)
        ninf = jnp.full((LANES,), -jnp.inf, jnp.float32)
        carry = (
            tuple(zero for _ in range(G)),
            tuple(pinf for _ in range(G)),
            tuple(ninf for _ in range(G)),
        )

        handles = [
            pltpu.async_copy(
                x_hbm.at[pl.ds(row0 + k * CH, CH)], bufs[k % 2], sems[k % 2]
            )
            for k in range(min(2, NCH))
        ]
        for k in range(NCH):
            cur = k % 2
            handles[cur].wait()
            xv = bufs[cur]

            def row_body(r, acc, xv=xv):
                sums, mns, mxs = acc
                new_s, new_n, new_x = [], [], []
                for g in range(G):
                    v = xv[r, pl.ds(g * LANES, LANES)]
                    new_s.append(sums[g] + v)
                    new_n.append(jnp.minimum(mns[g], v))
                    new_x.append(jnp.maximum(mxs[g], v))
                return (tuple(new_s), tuple(new_n), tuple(new_x))

            carry = lax.fori_loop(0, CH, row_body, carry)
            if k + 2 < NCH:
                handles[cur] = pltpu.async_copy(
                    x_hbm.at[pl.ds(row0 + (k + 2) * CH, CH)], bufs[cur], sems[cur]
                )

        sums, mns, mxs = carry
        for g in range(G):
            pv[pl.ds(g * LANES, LANES)] = sums[g]
            pv[pl.ds(D + g * LANES, LANES)] = mns[g]
            pv[pl.ds(2 * D + g * LANES, LANES)] = mxs[g]
        pltpu.sync_copy(pv, part_hbm.at[w // WPS, w % WPS])

    return _sc_stats


_sc_stats_calls = [_make_sc_stats(s * STAGE_B) for s in range(N_STAGES)]


def _asm_kernel(inv_ref, part_ref, x_ref, out_ref):
    i = pl.program_id(0)
    p = part_ref[0]
    inv = inv_ref[i]
    mean = jnp.sum(p[:, 0:D], axis=0, keepdims=True) * inv
    mn = jnp.min(p[:, D:2 * D], axis=0, keepdims=True)
    mx = jnp.max(p[:, 2 * D:3 * D], axis=0, keepdims=True)
    out_ref[:, 0:D] = jnp.broadcast_to(mean, (SEG, D))
    out_ref[:, D:2 * D] = jnp.broadcast_to(mn, (SEG, D))
    out_ref[:, 2 * D:3 * D] = jnp.broadcast_to(mx, (SEG, D))
    out_ref[:, 3 * D:4 * D] = x_ref[...]


def _xcopy_kernel(x_ref, out_ref):
    out_ref[...] = x_ref[...]


def _xcopy_call(x_data):
    return pl.pallas_call(
        _xcopy_kernel,
        grid=(B,),
        in_specs=[pl.BlockSpec((SEG, D), lambda i: (i, 0))],
        out_specs=pl.BlockSpec((SEG, D), lambda i: (i, 3)),
        out_shape=jax.ShapeDtypeStruct((TOTAL, 4 * D), jnp.float32),
    )(x_data)


def _stats_only_kernel(inv_ref, part_ref, buf_ref, out_ref):
    del buf_ref
    i = pl.program_id(0)
    p = part_ref[0]
    inv = inv_ref[i]
    mean = jnp.sum(p[:, 0:D], axis=0, keepdims=True) * inv
    mn = jnp.min(p[:, D:2 * D], axis=0, keepdims=True)
    mx = jnp.max(p[:, 2 * D:3 * D], axis=0, keepdims=True)
    out_ref[:, 0:D] = jnp.broadcast_to(mean, (SEG, D))
    out_ref[:, D:2 * D] = jnp.broadcast_to(mn, (SEG, D))
    out_ref[:, 2 * D:3 * D] = jnp.broadcast_to(mx, (SEG, D))


def _stats_only_call(seg_off, inv_half, part, buf):
    return pl.pallas_call(
        _stats_only_kernel,
        grid_spec=pltpu.PrefetchScalarGridSpec(
            num_scalar_prefetch=1,
            grid=(STAGE_B,),
            in_specs=[
                pl.BlockSpec((1, WPS, 3 * D), lambda i, *_: (i, 0, 0)),
                pl.BlockSpec(memory_space=pltpu.MemorySpace.HBM),
            ],
            out_specs=pl.BlockSpec((SEG, 3 * D), lambda i, *_: (i + seg_off, 0)),
        ),
        out_shape=jax.ShapeDtypeStruct((TOTAL, 4 * D), jnp.float32),
        input_output_aliases={2: 0},
    )(inv_half, part, buf)


def _asm_call(seg_off, inv_half, part, x_data, buf=None):
    in_specs = [
        pl.BlockSpec((1, WPS, 3 * D), lambda i, *_: (i, 0, 0)),
        pl.BlockSpec((SEG, D), lambda i, *_: (i + seg_off, 0)),
    ]
    operands = [inv_half, part, x_data]
    aliases = {}
    body = _asm_kernel
    if buf is not None:
        in_specs.append(pl.BlockSpec(memory_space=pltpu.MemorySpace.HBM))
        operands.append(buf)
        aliases = {3: 0}

        def body(inv_ref, part_ref, x_ref, buf_ref, out_ref):
            del buf_ref
            _asm_kernel(inv_ref, part_ref, x_ref, out_ref)

    return pl.pallas_call(
        body,
        grid_spec=pltpu.PrefetchScalarGridSpec(
            num_scalar_prefetch=1,
            grid=(STAGE_B,),
            in_specs=in_specs,
            out_specs=pl.BlockSpec((SEG, 4 * D), lambda i, *_: (i + seg_off, 0)),
        ),
        out_shape=jax.ShapeDtypeStruct((TOTAL, 4 * D), jnp.float32),
        input_output_aliases=aliases,
    )(*operands)


def kernel(x_data, row_splits):
    counts = (row_splits[1:] - row_splits[:-1]).astype(jnp.float32)
    inv_counts = 1.0 / counts
    parts = [sc(x_data) for sc in _sc_stats_calls]
    buf = _xcopy_call(x_data)
    buf = _stats_only_call(0, inv_counts[0:STAGE_B], parts[0], buf)
    return _stats_only_call(STAGE_B, inv_counts[STAGE_B:B], parts[1], buf)


# speedup vs baseline: 1.0477x; 1.0008x over previous
import functools

import jax
import jax.numpy as jnp
from jax import lax
from jax.experimental import pallas as pl
from jax.experimental.pallas import tpu as pltpu
from jax.experimental.pallas import tpu_sc as plsc

B = 16
TOTAL = 16384
D = 256
SEG = TOTAL // B
N_STAGES = 2
STAGE_B = B // N_STAGES
NC = 2
NS = 16
NW = NC * NS
WPS = NW // STAGE_B
ROWS_W = SEG // WPS
CH = 128
NCH = ROWS_W // CH
LANES = 16
G = D // LANES

_sc_mesh = plsc.VectorSubcoreMesh(core_axis_name="c", subcore_axis_name="s")


def _make_sc_stats(seg_off):
    row_off = seg_off * SEG

    @functools.partial(
        pl.kernel,
        out_type=jax.ShapeDtypeStruct((STAGE_B, WPS, 3 * D), jnp.float32),
        mesh=_sc_mesh,
        scratch_types=[
            pltpu.VMEM((CH, D), jnp.float32),
            pltpu.VMEM((CH, D), jnp.float32),
            pltpu.VMEM((3 * D,), jnp.float32),
            pltpu.SemaphoreType.DMA,
            pltpu.SemaphoreType.DMA,
        ],
    )
    def _sc_stats(x_hbm, part_hbm, xv0, xv1, pv, sem0, sem1):
        c = lax.axis_index("c")
        sub = lax.axis_index("s")
        w = c * NS + sub
        row0 = row_off + w * ROWS_W

        bufs = (xv0, xv1)
        sems = (sem0, sem1)

        zero = jnp.zeros((LANES,), jnp.float32)
        pinf = jnp.full((LANES,), jnp.inf, jnp.float32)
        ninf = jnp.full((LANES,), -jnp.inf, jnp.float32)
        carry = (
            tuple(zero for _ in range(G)),
            tuple(pinf for _ in range(G)),
            tuple(ninf for _ in range(G)),
        )

        handles = [
            pltpu.async_copy(
                x_hbm.at[pl.ds(row0 + k * CH, CH)], bufs[k % 2], sems[k % 2]
            )
            for k in range(min(2, NCH))
        ]
        for k in range(NCH):
            cur = k % 2
            handles[cur].wait()
            xv = bufs[cur]

            def row_body(r, acc, xv=xv):
                sums, mns, mxs = acc
                new_s, new_n, new_x = [], [], []
                for g in range(G):
                    v = xv[r, pl.ds(g * LANES, LANES)]
                    new_s.append(sums[g] + v)
                    new_n.append(jnp.minimum(mns[g], v))
                    new_x.append(jnp.maximum(mxs[g], v))
                return (tuple(new_s), tuple(new_n), tuple(new_x))

            carry = lax.fori_loop(0, CH, row_body, carry)
            if k + 2 < NCH:
                handles[cur] = pltpu.async_copy(
                    x_hbm.at[pl.ds(row0 + (k + 2) * CH, CH)], bufs[cur], sems[cur]
                )

        sums, mns, mxs = carry
        for g in range(G):
            pv[pl.ds(g * LANES, LANES)] = sums[g]
            pv[pl.ds(D + g * LANES, LANES)] = mns[g]
            pv[pl.ds(2 * D + g * LANES, LANES)] = mxs[g]
        pltpu.sync_copy(pv, part_hbm.at[w // WPS, w % WPS])

    return _sc_stats


_sc_stats_calls = [_make_sc_stats(s * STAGE_B) for s in range(N_STAGES)]


def _xcopy_kernel(x_ref, out_ref):
    out_ref[...] = x_ref[...]


def _xcopy_call(x_data):
    return pl.pallas_call(
        _xcopy_kernel,
        grid=(B,),
        in_specs=[pl.BlockSpec((SEG, D), lambda i: (i, 0))],
        out_specs=pl.BlockSpec((SEG, D), lambda i: (i, 3)),
        out_shape=jax.ShapeDtypeStruct((TOTAL, 4 * D), jnp.float32),
    )(x_data)


def _stats_only_kernel(inv_ref, part_ref, buf_ref, out_ref):
    del buf_ref
    i = pl.program_id(0)
    p = part_ref[0]
    inv = inv_ref[i]
    mean = jnp.sum(p[:, 0:D], axis=0, keepdims=True) * inv
    mn = jnp.min(p[:, D:2 * D], axis=0, keepdims=True)
    mx = jnp.max(p[:, 2 * D:3 * D], axis=0, keepdims=True)
    out_ref[:, 0:D] = jnp.broadcast_to(mean, (SEG, D))
    out_ref[:, D:2 * D] = jnp.broadcast_to(mn, (SEG, D))
    out_ref[:, 2 * D:3 * D] = jnp.broadcast_to(mx, (SEG, D))


def _stats_only_call(seg_off, inv_half, part, buf):
    return pl.pallas_call(
        _stats_only_kernel,
        grid_spec=pltpu.PrefetchScalarGridSpec(
            num_scalar_prefetch=1,
            grid=(STAGE_B,),
            in_specs=[
                pl.BlockSpec((1, WPS, 3 * D), lambda i, *_: (i, 0, 0)),
                pl.BlockSpec(memory_space=pltpu.MemorySpace.HBM),
            ],
            out_specs=pl.BlockSpec((SEG, 3 * D), lambda i, *_: (i + seg_off, 0)),
        ),
        out_shape=jax.ShapeDtypeStruct((TOTAL, 4 * D), jnp.float32),
        input_output_aliases={2: 0},
    )(inv_half, part, buf)


def kernel(x_data, row_splits):
    counts = (row_splits[1:] - row_splits[:-1]).astype(jnp.float32)
    inv_counts = 1.0 / counts
    parts = [sc(x_data) for sc in _sc_stats_calls]
    buf = _xcopy_call(x_data)
    buf = _stats_only_call(0, inv_counts[0:STAGE_B], parts[0], buf)
    return _stats_only_call(STAGE_B, inv_counts[STAGE_B:B], parts[1], buf)


# speedup vs baseline: 1.0676x; 1.0190x over previous
import functools

import jax
import jax.numpy as jnp
from jax import lax
from jax.experimental import pallas as pl
from jax.experimental.pallas import tpu as pltpu
from jax.experimental.pallas import tpu_sc as plsc

B = 16
TOTAL = 16384
D = 256
SEG = TOTAL // B
N_STAGES = 2
STAGE_B = B // N_STAGES
NC = 2
NS = 16
NW = NC * NS
WPS = NW // STAGE_B
ROWS_W = SEG // WPS
CH = 128
NCH = ROWS_W // CH
LANES = 16
G = D // LANES

_sc_mesh = plsc.VectorSubcoreMesh(core_axis_name="c", subcore_axis_name="s")


def _make_sc_stats(seg_off):
    row_off = seg_off * SEG

    @functools.partial(
        pl.kernel,
        out_type=jax.ShapeDtypeStruct((STAGE_B, WPS, 3 * D), jnp.float32),
        mesh=_sc_mesh,
        scratch_types=[
            pltpu.VMEM((CH, D), jnp.float32),
            pltpu.VMEM((CH, D), jnp.float32),
            pltpu.VMEM((3 * D,), jnp.float32),
            pltpu.SemaphoreType.DMA,
            pltpu.SemaphoreType.DMA,
        ],
    )
    def _sc_stats(x_hbm, part_hbm, xv0, xv1, pv, sem0, sem1):
        c = lax.axis_index("c")
        sub = lax.axis_index("s")
        w = c * NS + sub
        row0 = row_off + w * ROWS_W

        bufs = (xv0, xv1)
        sems = (sem0, sem1)

        zero = jnp.zeros((LANES,), jnp.float32)
        pinf = jnp.full((LANES,), jnp.inf, jnp.float32)
        ninf = jnp.full((LANES,), -jnp.inf, jnp.float32)
        carry = (
            tuple(zero for _ in range(G)),
            tuple(pinf for _ in range(G)),
            tuple(ninf for _ in range(G)),
        )

        handles = [
            pltpu.async_copy(
                x_hbm.at[pl.ds(row0 + k * CH, CH)], bufs[k % 2], sems[k % 2]
            )
            for k in range(min(2, NCH))
        ]
        for k in range(NCH):
            cur = k % 2
            handles[cur].wait()
            xv = bufs[cur]

            def row_body(r, acc, xv=xv):
                sums, mns, mxs = acc
                new_s, new_n, new_x = [], [], []
                for g in range(G):
                    v = xv[r, pl.ds(g * LANES, LANES)]
                    new_s.append(sums[g] + v)
                    new_n.append(jnp.minimum(mns[g], v))
                    new_x.append(jnp.maximum(mxs[g], v))
                return (tuple(new_s), tuple(new_n), tuple(new_x))

            carry = lax.fori_loop(0, CH, row_body, carry)
            if k + 2 < NCH:
                handles[cur] = pltpu.async_copy(
                    x_hbm.at[pl.ds(row0 + (k + 2) * CH, CH)], bufs[cur], sems[cur]
                )

        sums, mns, mxs = carry
        for g in range(G):
            pv[pl.ds(g * LANES, LANES)] = sums[g]
            pv[pl.ds(D + g * LANES, LANES)] = mns[g]
            pv[pl.ds(2 * D + g * LANES, LANES)] = mxs[g]
        pltpu.sync_copy(pv, part_hbm.at[w // WPS, w % WPS])

    return _sc_stats


_sc_stats_calls = [_make_sc_stats(s * STAGE_B) for s in range(N_STAGES)]


def _xcopy_kernel(x_ref, out_ref):
    out_ref[...] = x_ref[...]


def _xcopy_call(x_data):
    return pl.pallas_call(
        _xcopy_kernel,
        grid=(B,),
        in_specs=[pl.BlockSpec((SEG, D), lambda i: (i, 0))],
        out_specs=pl.BlockSpec((SEG, D), lambda i: (i, 3)),
        out_shape=jax.ShapeDtypeStruct((TOTAL, 4 * D), jnp.float32),
    )(x_data)


def _make_stats_only_kernel(seg_off):
    def _stats_only_kernel(splits_ref, part_ref, buf_ref, out_ref):
        del buf_ref
        i = pl.program_id(0)
        s = i + seg_off
        p = part_ref[0]
        count = (splits_ref[s + 1] - splits_ref[s]).astype(jnp.float32)
        mean = jnp.sum(p[:, 0:D], axis=0, keepdims=True) * (1.0 / count)
        mn = jnp.min(p[:, D:2 * D], axis=0, keepdims=True)
        mx = jnp.max(p[:, 2 * D:3 * D], axis=0, keepdims=True)
        out_ref[:, 0:D] = jnp.broadcast_to(mean, (SEG, D))
        out_ref[:, D:2 * D] = jnp.broadcast_to(mn, (SEG, D))
        out_ref[:, 2 * D:3 * D] = jnp.broadcast_to(mx, (SEG, D))

    return _stats_only_kernel


def _stats_only_call(seg_off, row_splits, part, buf):
    return pl.pallas_call(
        _make_stats_only_kernel(seg_off),
        grid_spec=pltpu.PrefetchScalarGridSpec(
            num_scalar_prefetch=1,
            grid=(STAGE_B,),
            in_specs=[
                pl.BlockSpec((1, WPS, 3 * D), lambda i, *_: (i, 0, 0)),
                pl.BlockSpec(memory_space=pltpu.MemorySpace.HBM),
            ],
            out_specs=pl.BlockSpec((SEG, 3 * D), lambda i, *_: (i + seg_off, 0)),
        ),
        out_shape=jax.ShapeDtypeStruct((TOTAL, 4 * D), jnp.float32),
        input_output_aliases={2: 0},
    )(row_splits, part, buf)


def kernel(x_data, row_splits):
    parts = [sc(x_data) for sc in _sc_stats_calls]
    buf = _xcopy_call(x_data)
    buf = _stats_only_call(0, row_splits, parts[0], buf)
    return _stats_only_call(STAGE_B, row_splits, parts[1], buf)
